# baseline (speedup 1.0000x reference)
import functools

import jax
import jax.numpy as jnp
from jax import lax
from jax.experimental import pallas as pl
from jax.experimental.pallas import tpu as pltpu
from jax.experimental.pallas import tpu_sc as plsc

_SMOOTHING = 0.1
_IGNORE_INDEX = 0
_CONFIDENCE = 1.0 - _SMOOTHING

_BC = 2256
_SC_CLS = 23296
_CPW = _SC_CLS // 32
_CK = 56
_TPS = 3


def _col_stats_kernel(predt_ref, m_ref, s_ref, t_ref):
    i = pl.program_id(0)
    x = predt_ref[:, :]
    bc, n = x.shape
    g = 8
    xr = x.reshape(bc // g, g, n)

    @pl.when(i == 0)
    def _init():
        m_ref[:, :] = jnp.full((g, n), -jnp.inf, x.dtype)
        s_ref[:, :] = jnp.zeros((g, n), x.dtype)
        t_ref[:, :] = jnp.zeros((g, n), x.dtype)

    m_blk = jnp.max(xr, axis=0)
    m_old = m_ref[:, :]
    m_new = jnp.maximum(m_old, m_blk)
    s_new = s_ref[:, :] * jnp.exp(m_old - m_new) + jnp.sum(
        jnp.exp(xr - m_new[None]), axis=0)
    m_ref[:, :] = m_new
    s_ref[:, :] = s_new
    t_ref[:, :] = t_ref[:, :] + jnp.sum(xr, axis=0)


def _sc_cls_kernel(predt_hbm, tgt_hbm, out_hbm,
                   buf0, buf1, macc, sacc, tocc, stage, tgt_v, xt_v,
                   sem0, sem1, semg, *, c1, n_tok, chunks):
    wid = lax.axis_index("c") * 16 + lax.axis_index("s")
    cls0 = pl.multiple_of(c1 + wid * _CPW, 8)
    n_chunks = len(chunks)
    n_tg = n_tok // 16
    bufs = (buf0, buf1)
    sems = (sem0, sem1)

    pltpu.sync_copy(tgt_hbm.at[pl.ds(pl.multiple_of(wid * 32, 8), 32)], tgt_v)
    tvA = tgt_v[pl.ds(0, 16)]
    tvB = tgt_v[pl.ds(16, 16)]

    def _start(step):
        o, ck = chunks[step]
        pltpu.make_async_copy(
            predt_hbm.at[pl.ds(cls0 + o, ck), :],
            bufs[step % 2].at[pl.ds(0, ck), :], sems[step % 2]).start()

    iota16 = lax.iota(jnp.int32, 16)
    neg_inf = jnp.full((16,), -jnp.inf, jnp.float32)
    zeros = jnp.zeros((16,), jnp.float32)

    xt = [zeros, zeros]

    def _tok_params(j):
        t_s = (tvA if j < 16 else tvB)[j % 16]
        t8 = pl.multiple_of((t_s // 8) * 8, 8)
        tok = wid * 32 + j
        tile0 = pl.multiple_of((tok // 128) * 128, 128)
        return t_s, t8, tile0, tok

    def _stage_copy(j, slot):
        t_s, t8, tile0, _ = _tok_params(j)
        return pltpu.make_async_copy(
            predt_hbm.at[pl.ds(t8, 8), pl.ds(tile0, 128)],
            stage.at[pl.ds(slot * 8, 8), :], semg)

    def _stage_fin(j, slot):
        _stage_copy(j, slot).wait()
        t_s, t8, tile0, tok = _tok_params(j)
        lane = tok - tile0
        l16 = pl.multiple_of((lane // 16) * 16, 16)
        li = lane % 16
        rr = t_s - t8
        ev = zeros
        for r in range(8):
            v = stage[slot * 8 + r, pl.ds(l16, 16)]
            ev = ev + jnp.where((iota16 == li) & (rr == r), v, 0.0)
        es = jnp.sum(ev)
        xt[j // 16] = jnp.where(iota16 == j % 16, jnp.full((16,), es),
                                xt[j // 16])

    def _init_tg(tg, carry):
        macc[pl.ds(tg * 16, 16)] = neg_inf
        sacc[pl.ds(tg * 16, 16)] = zeros
        tocc[pl.ds(tg * 16, 16)] = zeros
        return carry
    lax.fori_loop(0, n_tg, _init_tg, 0)

    _start(0)
    for step in range(n_chunks):
        o, ck = chunks[step]
        buf = bufs[step % 2]
        pltpu.make_async_copy(
            predt_hbm.at[pl.ds(cls0 + o, ck), :],
            buf.at[pl.ds(0, ck), :], sems[step % 2]).wait()
        if step + 1 < n_chunks:
            _start(step + 1)
        for k in range(_TPS):
            if step * _TPS + k < 32:
                _stage_copy(step * _TPS + k, k).start()

        def _do_tg(tg, carry, buf=buf, ck=ck):
            mv = macc[pl.ds(tg * 16, 16)]
            tv = tocc[pl.ds(tg * 16, 16)]

            def _sweep1(i, carry):
                cm, tr = carry
                v = buf[i, pl.ds(tg * 16, 16)]
                return jnp.maximum(cm, v), tr + v
            cm, tv = lax.fori_loop(0, ck, _sweep1, (mv, tv), unroll=8)

            sv = sacc[pl.ds(tg * 16, 16)] * jnp.exp(mv - cm)

            def _sweep2(i, sr):
                v = buf[i, pl.ds(tg * 16, 16)]
                return sr + jnp.exp(v - cm)
            sv = lax.fori_loop(0, ck, _sweep2, sv, unroll=8)

            macc[pl.ds(tg * 16, 16)] = cm
            sacc[pl.ds(tg * 16, 16)] = sv
            tocc[pl.ds(tg * 16, 16)] = tv
            return carry
        lax.fori_loop(0, n_tg, _do_tg, 0)

        for k in range(_TPS):
            if step * _TPS + k < 32:
                _stage_fin(step * _TPS + k, k)

    base = wid * (3 * n_tok + 32)
    pltpu.sync_copy(macc, out_hbm.at[pl.ds(base, n_tok)])
    pltpu.sync_copy(sacc, out_hbm.at[pl.ds(base + n_tok, n_tok)])
    pltpu.sync_copy(tocc, out_hbm.at[pl.ds(base + 2 * n_tok, n_tok)])
    xt_v[pl.ds(0, 16)] = xt[0]
    xt_v[pl.ds(16, 16)] = xt[1]
    pltpu.sync_copy(xt_v, out_hbm.at[pl.ds(base + 3 * n_tok, 32)])


def kernel(pred, target):
    n, c = pred.shape
    predt = pred.T
    tgt = target.astype(jnp.int32)

    c1 = c - _SC_CLS
    chunks = []
    o = 0
    while o < _CPW:
        ck = min(_CK, _CPW - o)
        chunks.append((o, ck))
        o += ck

    wlen = 3 * n + 32
    sc_out = pl.kernel(
        functools.partial(_sc_cls_kernel, c1=c1, n_tok=n,
                          chunks=tuple(chunks)),
        out_type=jax.ShapeDtypeStruct((32 * wlen,), jnp.float32),
        mesh=plsc.VectorSubcoreMesh(core_axis_name="c", subcore_axis_name="s"),
        compiler_params=pltpu.CompilerParams(needs_layout_passes=False),
        scratch_types=[
            pltpu.VMEM((_CK, n), jnp.float32),
            pltpu.VMEM((_CK, n), jnp.float32),
            pltpu.VMEM((n,), jnp.float32),
            pltpu.VMEM((n,), jnp.float32),
            pltpu.VMEM((n,), jnp.float32),
            pltpu.VMEM((_TPS * 8, 128), jnp.float32),
            pltpu.VMEM((32,), jnp.int32),
            pltpu.VMEM((32,), jnp.float32),
            pltpu.SemaphoreType.DMA,
            pltpu.SemaphoreType.DMA,
            pltpu.SemaphoreType.DMA,
        ],
    )(predt, tgt)

    g = 8
    m8, s8, t8 = pl.pallas_call(
        _col_stats_kernel,
        grid=(c1 // _BC,),
        in_specs=[pl.BlockSpec((_BC, n), lambda i: (i, 0))],
        out_specs=[pl.BlockSpec((g, n), lambda i: (0, 0))] * 3,
        out_shape=[jax.ShapeDtypeStruct((g, n), pred.dtype)] * 3,
    )(predt)

    sc = sc_out.reshape(32, wlen)
    m_p = sc[:, :n]
    s_p = sc[:, n:2 * n]
    t_p = sc[:, 2 * n:3 * n]
    xt = sc[:, 3 * n:].reshape(-1)

    m_tc = jnp.max(m8, axis=0)
    m_all = jnp.maximum(m_tc, jnp.max(m_p, axis=0))
    s_all = (jnp.sum(s8 * jnp.exp(m8 - m_all[None]), axis=0)
             + jnp.sum(s_p * jnp.exp(m_p - m_all[None]), axis=0))
    tot = jnp.sum(t8, axis=0) + jnp.sum(t_p, axis=0)
    lse = m_all + jnp.log(s_all)

    fill = _SMOOTHING / (c - 1) if c > 1 else _SMOOTHING
    pt = -(fill * (tot - c * lse) + (_CONFIDENCE - fill) * (xt - lse))
    keep = (tgt != _IGNORE_INDEX).astype(pred.dtype)
    return jnp.sum(pt * keep) / jnp.maximum(jnp.sum(keep), 1.0)


# speedup vs baseline: 1.0801x; 1.0801x over previous
import functools

import jax
import jax.numpy as jnp
from jax import lax
from jax.experimental import pallas as pl
from jax.experimental.pallas import tpu as pltpu
from jax.experimental.pallas import tpu_sc as plsc

_SMOOTHING = 0.1
_IGNORE_INDEX = 0
_CONFIDENCE = 1.0 - _SMOOTHING

_BC = 4512
_SC_CLS = 23296
_CPW = _SC_CLS // 32
_CK = 56
_TPS = 3


def _col_stats_kernel(predt_ref, m_ref, s_ref, t_ref):
    i = pl.program_id(0)
    x = predt_ref[:, :]
    bc, n = x.shape
    g = 8
    xr = x.reshape(bc // g, g, n)

    @pl.when(i == 0)
    def _init():
        m_ref[:, :] = jnp.full((g, n), -jnp.inf, x.dtype)
        s_ref[:, :] = jnp.zeros((g, n), x.dtype)
        t_ref[:, :] = jnp.zeros((g, n), x.dtype)

    m_blk = jnp.max(xr, axis=0)
    m_old = m_ref[:, :]
    m_new = jnp.maximum(m_old, m_blk)
    s_new = s_ref[:, :] * jnp.exp(m_old - m_new) + jnp.sum(
        jnp.exp(xr - m_new[None]), axis=0)
    m_ref[:, :] = m_new
    s_ref[:, :] = s_new
    t_ref[:, :] = t_ref[:, :] + jnp.sum(xr, axis=0)


def _sc_cls_kernel(predt_hbm, tgt_hbm, out_hbm,
                   buf0, buf1, macc, sacc, tocc, stage, tgt_v, xt_v,
                   sem0, sem1, semg, *, c1, n_tok, chunks):
    wid = lax.axis_index("c") * 16 + lax.axis_index("s")
    cls0 = pl.multiple_of(c1 + wid * _CPW, 8)
    n_chunks = len(chunks)
    n_tg = n_tok // 16
    bufs = (buf0, buf1)
    sems = (sem0, sem1)

    pltpu.sync_copy(tgt_hbm.at[pl.ds(pl.multiple_of(wid * 32, 8), 32)], tgt_v)
    tvA = tgt_v[pl.ds(0, 16)]
    tvB = tgt_v[pl.ds(16, 16)]

    def _start(step):
        o, ck = chunks[step]
        pltpu.make_async_copy(
            predt_hbm.at[pl.ds(cls0 + o, ck), :],
            bufs[step % 2].at[pl.ds(0, ck), :], sems[step % 2]).start()

    iota16 = lax.iota(jnp.int32, 16)
    neg_inf = jnp.full((16,), -jnp.inf, jnp.float32)
    zeros = jnp.zeros((16,), jnp.float32)

    xt = [zeros, zeros]

    def _tok_params(j):
        t_s = (tvA if j < 16 else tvB)[j % 16]
        t8 = pl.multiple_of((t_s // 8) * 8, 8)
        tok = wid * 32 + j
        tile0 = pl.multiple_of((tok // 128) * 128, 128)
        return t_s, t8, tile0, tok

    def _stage_copy(j, slot):
        t_s, t8, tile0, _ = _tok_params(j)
        return pltpu.make_async_copy(
            predt_hbm.at[pl.ds(t8, 8), pl.ds(tile0, 128)],
            stage.at[pl.ds(slot * 8, 8), :], semg)

    def _stage_fin(j, slot):
        _stage_copy(j, slot).wait()
        t_s, t8, tile0, tok = _tok_params(j)
        lane = tok - tile0
        l16 = pl.multiple_of((lane // 16) * 16, 16)
        li = lane % 16
        rr = t_s - t8
        ev = zeros
        for r in range(8):
            v = stage[slot * 8 + r, pl.ds(l16, 16)]
            ev = ev + jnp.where((iota16 == li) & (rr == r), v, 0.0)
        es = jnp.sum(ev)
        xt[j // 16] = jnp.where(iota16 == j % 16, jnp.full((16,), es),
                                xt[j // 16])

    def _init_tg(tg, carry):
        macc[pl.ds(tg * 16, 16)] = neg_inf
        sacc[pl.ds(tg * 16, 16)] = zeros
        tocc[pl.ds(tg * 16, 16)] = zeros
        return carry
    lax.fori_loop(0, n_tg, _init_tg, 0)

    _start(0)
    for step in range(n_chunks):
        o, ck = chunks[step]
        buf = bufs[step % 2]
        pltpu.make_async_copy(
            predt_hbm.at[pl.ds(cls0 + o, ck), :],
            buf.at[pl.ds(0, ck), :], sems[step % 2]).wait()
        if step + 1 < n_chunks:
            _start(step + 1)
        for k in range(_TPS):
            if step * _TPS + k < 32:
                _stage_copy(step * _TPS + k, k).start()

        def _do_tg(tg, carry, buf=buf, ck=ck):
            mv = macc[pl.ds(tg * 16, 16)]
            tv = tocc[pl.ds(tg * 16, 16)]

            def _sweep1(i, carry):
                cm, tr = carry
                v = buf[i, pl.ds(tg * 16, 16)]
                return jnp.maximum(cm, v), tr + v
            cm, tv = lax.fori_loop(0, ck, _sweep1, (mv, tv), unroll=8)

            sv = sacc[pl.ds(tg * 16, 16)] * jnp.exp(mv - cm)

            def _sweep2(i, sr):
                v = buf[i, pl.ds(tg * 16, 16)]
                return sr + jnp.exp(v - cm)
            sv = lax.fori_loop(0, ck, _sweep2, sv, unroll=8)

            macc[pl.ds(tg * 16, 16)] = cm
            sacc[pl.ds(tg * 16, 16)] = sv
            tocc[pl.ds(tg * 16, 16)] = tv
            return carry
        lax.fori_loop(0, n_tg, _do_tg, 0)

        for k in range(_TPS):
            if step * _TPS + k < 32:
                _stage_fin(step * _TPS + k, k)

    base = wid * (3 * n_tok + 32)
    pltpu.sync_copy(macc, out_hbm.at[pl.ds(base, n_tok)])
    pltpu.sync_copy(sacc, out_hbm.at[pl.ds(base + n_tok, n_tok)])
    pltpu.sync_copy(tocc, out_hbm.at[pl.ds(base + 2 * n_tok, n_tok)])
    xt_v[pl.ds(0, 16)] = xt[0]
    xt_v[pl.ds(16, 16)] = xt[1]
    pltpu.sync_copy(xt_v, out_hbm.at[pl.ds(base + 3 * n_tok, 32)])


def kernel(pred, target):
    n, c = pred.shape
    predt = pred.T
    tgt = target.astype(jnp.int32)

    c1 = c - _SC_CLS
    chunks = []
    o = 0
    while o < _CPW:
        ck = min(_CK, _CPW - o)
        chunks.append((o, ck))
        o += ck

    wlen = 3 * n + 32
    sc_out = pl.kernel(
        functools.partial(_sc_cls_kernel, c1=c1, n_tok=n,
                          chunks=tuple(chunks)),
        out_type=jax.ShapeDtypeStruct((32 * wlen,), jnp.float32),
        mesh=plsc.VectorSubcoreMesh(core_axis_name="c", subcore_axis_name="s"),
        compiler_params=pltpu.CompilerParams(needs_layout_passes=False),
        scratch_types=[
            pltpu.VMEM((_CK, n), jnp.float32),
            pltpu.VMEM((_CK, n), jnp.float32),
            pltpu.VMEM((n,), jnp.float32),
            pltpu.VMEM((n,), jnp.float32),
            pltpu.VMEM((n,), jnp.float32),
            pltpu.VMEM((_TPS * 8, 128), jnp.float32),
            pltpu.VMEM((32,), jnp.int32),
            pltpu.VMEM((32,), jnp.float32),
            pltpu.SemaphoreType.DMA,
            pltpu.SemaphoreType.DMA,
            pltpu.SemaphoreType.DMA,
        ],
    )(predt, tgt)

    g = 8
    m8, s8, t8 = pl.pallas_call(
        _col_stats_kernel,
        grid=(c1 // _BC,),
        in_specs=[pl.BlockSpec((_BC, n), lambda i: (i, 0))],
        out_specs=[pl.BlockSpec((g, n), lambda i: (0, 0))] * 3,
        out_shape=[jax.ShapeDtypeStruct((g, n), pred.dtype)] * 3,
    )(predt)

    sc = sc_out.reshape(32, wlen)
    m_p = sc[:, :n]
    s_p = sc[:, n:2 * n]
    t_p = sc[:, 2 * n:3 * n]
    xt = sc[:, 3 * n:].reshape(-1)

    m_tc = jnp.max(m8, axis=0)
    m_all = jnp.maximum(m_tc, jnp.max(m_p, axis=0))
    s_all = (jnp.sum(s8 * jnp.exp(m8 - m_all[None]), axis=0)
             + jnp.sum(s_p * jnp.exp(m_p - m_all[None]), axis=0))
    tot = jnp.sum(t8, axis=0) + jnp.sum(t_p, axis=0)
    lse = m_all + jnp.log(s_all)

    fill = _SMOOTHING / (c - 1) if c > 1 else _SMOOTHING
    pt = -(fill * (tot - c * lse) + (_CONFIDENCE - fill) * (xt - lse))
    keep = (tgt != _IGNORE_INDEX).astype(pred.dtype)
    return jnp.sum(pt * keep) / jnp.maximum(jnp.sum(keep), 1.0)
